# Initial kernel scaffold; baseline (speedup 1.0000x reference)
import jax
import jax.numpy as jnp
from jax.experimental import pallas as pl


def kernel(queries, ref_points, value, value_spatial_shapes, W_v, W_off, b_off, W_attn, b_attn, W_out):
    raise NotImplementedError("write your pallas kernel here")


# speedup vs baseline: 222.3650x; 222.3650x over previous
import functools

import jax
import jax.numpy as jnp
from jax.experimental import pallas as pl
from jax.experimental.pallas import tpu as pltpu
from jax.experimental.pallas import tpu_sc as plsc

_N, _Q, _D = 1, 900, 256
_H, _V, _L, _R, _Z = 8, 6, 4, 4, 4
_HD = _D // _H
_SHAPES = [(64, 64), (32, 32), (16, 16), (8, 8)]
_S = sum(h * w for h, w in _SHAPES)
_STARTS = [0, 4096, 5120, 5376]
_QP = 1024
_QB = _QP // 128
_VL = _V * _L
_RZ = _R * _Z
_HRZ = _H * _RZ
_NROWS = _VL * _QP * _HRZ
_CHUNK = 128


def _vproj_kernel(val_ref, wv_ref, out_ref):
    out_ref[...] = jnp.dot(val_ref[...], wv_ref[...],
                           preferred_element_type=jnp.float32)


def _qproj_kernel(q_ref, woff_ref, boff_ref, wattn_ref, battn_ref,
                  off_ref, attn_ref):
    q = q_ref[...]
    off = jnp.dot(q, woff_ref[...], preferred_element_type=jnp.float32)
    off_ref[...] = jnp.tanh(off + boff_ref[...])
    a = jnp.dot(q, wattn_ref[...], preferred_element_type=jnp.float32)
    a = a + battn_ref[...]
    vlrz = _V * _L * _R * _Z
    for h in range(_H):
        ah = a[:, h * vlrz:(h + 1) * vlrz]
        m = jnp.max(ah, axis=-1, keepdims=True)
        e = jnp.exp(ah - m)
        attn_ref[:, h * vlrz:(h + 1) * vlrz] = e / jnp.sum(e, axis=-1,
                                                           keepdims=True)


def _addr_kernel(offx_ref, offy_ref, attn_ref, refx_ref, refy_ref,
                 lin_ref, w00_ref, w01_ref, w10_ref, w11_ref):
    vl = pl.program_id(0)
    l = vl % _L
    v = vl // _L
    wl = jax.lax.shift_right_logical(jnp.int32(64), l)
    wlf = wl.astype(jnp.float32)
    start = jnp.where(
        l == 0, 0, jnp.where(l == 1, _STARTS[1],
                             jnp.where(l == 2, _STARTS[2], _STARTS[3])))

    refx = refx_ref[0]
    refy = refy_ref[0]
    rtx = jnp.concatenate([refx] * _R, axis=1)
    rty = jnp.concatenate([refy] * _R, axis=1)
    rfx = jnp.concatenate([rtx] * _H, axis=1)
    rfy = jnp.concatenate([rty] * _H, axis=1)

    ox = offx_ref[0]
    oy = offy_ref[0]
    at = attn_ref[0]

    gx = (jnp.clip(rfx + ox, -1.0, 1.0) + 1.0) * 0.5 * (wlf - 1.0)
    gy = (jnp.clip(rfy + oy, -1.0, 1.0) + 1.0) * 0.5 * (wlf - 1.0)
    x0 = jnp.floor(gx)
    y0 = jnp.floor(gy)
    wx1 = gx - x0
    wx0 = 1.0 - wx1
    wy1 = gy - y0
    wy0 = 1.0 - wy1
    ix0 = x0.astype(jnp.int32)
    iy0 = y0.astype(jnp.int32)

    hlane = jax.lax.shift_right_logical(
        jax.lax.broadcasted_iota(jnp.int32, (128, _HRZ), 1), 4)
    lin_ref[0] = (v * _S + start + iy0 * wl + ix0) * _H + hlane
    w00_ref[0] = at * wy0 * wx0
    w01_ref[0] = at * wy0 * wx1
    w10_ref[0] = at * wy1 * wx0
    w11_ref[0] = at * wy1 * wx1


def _sc_gather_body(table_hbm, idx_hbm, out_hbm, idx_v, rows_v, sem):
    info = plsc.get_sparse_core_info()
    nw = info.num_cores * info.num_subcores
    per_w = _NROWS // nw
    n_chunks = per_w // _CHUNK
    wid = jax.lax.axis_index("s") * info.num_cores + jax.lax.axis_index("c")

    def body(i, carry):
        base = wid * per_w + i * _CHUNK
        pltpu.sync_copy(idx_hbm.at[pl.ds(base, _CHUNK)], idx_v)
        pltpu.async_copy(table_hbm.at[idx_v], rows_v, sem).wait()
        pltpu.sync_copy(rows_v, out_hbm.at[pl.ds(base, _CHUNK)])
        return carry

    jax.lax.fori_loop(0, n_chunks, body, 0)


def _sc_gather_rows(table, idx):
    kfn = functools.partial(
        pl.kernel,
        out_type=jax.ShapeDtypeStruct((_NROWS, 4 * _HD), jnp.float32),
        mesh=plsc.VectorSubcoreMesh(core_axis_name="c", subcore_axis_name="s"),
        scratch_types=[
            pltpu.VMEM((_CHUNK,), jnp.int32),
            pltpu.VMEM((_CHUNK, 4 * _HD), jnp.float32),
            pltpu.SemaphoreType.DMA,
        ],
    )(_sc_gather_body)
    return kfn(table, idx)


def _reduce_kernel(rows_ref, wt_ref, out_ref):
    rows = rows_ref[...]
    wt = wt_ref[...]
    for h in range(_H):
        rh = rows[:, :, h]
        wh = wt[:, :, h]
        acc = wh[..., 0:1] * rh[..., 0:_HD] \
            + wh[..., 1:2] * rh[..., _HD:2 * _HD] \
            + wh[..., 2:3] * rh[..., 2 * _HD:3 * _HD] \
            + wh[..., 3:4] * rh[..., 3 * _HD:4 * _HD]
        red = acc.sum(axis=0).sum(axis=1)
        out_ref[:, h * _HD:(h + 1) * _HD] = red


def _oproj_kernel(acc_ref, wout_ref, out_ref):
    out_ref[...] = jnp.dot(acc_ref[...], wout_ref[...],
                           preferred_element_type=jnp.float32)


def kernel(queries, ref_points, value, value_spatial_shapes,
           W_v, W_off, b_off, W_attn, b_attn, W_out):
    f32 = jnp.float32
    q2 = queries.reshape(_Q, _D)
    qpad = jnp.pad(q2, ((0, _QP - _Q), (0, 0)))

    vflat = value.reshape(_V * _S, _D)
    n_vrows = (_V * _S) // 128
    vproj = pl.pallas_call(
        _vproj_kernel,
        grid=(n_vrows,),
        in_specs=[
            pl.BlockSpec((128, _D), lambda i: (i, 0)),
            pl.BlockSpec((_D, _D), lambda i: (0, 0)),
        ],
        out_specs=pl.BlockSpec((128, _D), lambda i: (i, 0)),
        out_shape=jax.ShapeDtypeStruct((_V * _S, _D), f32),
    )(vflat, W_v.T)
    vp3 = vproj.reshape(_V, _S, _H, _HD)
    parts = []
    s0 = 0
    for (hl, wl_) in _SHAPES:
        fm = vp3[:, s0:s0 + hl * wl_].reshape(_V, hl, wl_, _H, _HD)
        fx = jnp.concatenate([fm[:, :, 1:], fm[:, :, -1:]], axis=2)
        fy = jnp.concatenate([fm[:, 1:], fm[:, -1:]], axis=1)
        fxy = jnp.concatenate([fy[:, :, 1:], fy[:, :, -1:]], axis=2)
        p = jnp.stack([fm, fx, fy, fxy], axis=4)
        parts.append(p.reshape(_V, hl * wl_, _H, 4 * _HD))
        s0 += hl * wl_
    table = jnp.concatenate(parts, axis=1).reshape(_V * _S * _H, 4 * _HD)

    aout = _H * _V * _L * _R * _Z
    offs, attn = pl.pallas_call(
        _qproj_kernel,
        grid=(_QB,),
        in_specs=[
            pl.BlockSpec((128, _D), lambda i: (i, 0)),
            pl.BlockSpec((_D, aout * 2), lambda i: (0, 0)),
            pl.BlockSpec((1, aout * 2), lambda i: (0, 0)),
            pl.BlockSpec((_D, aout), lambda i: (0, 0)),
            pl.BlockSpec((1, aout), lambda i: (0, 0)),
        ],
        out_specs=[
            pl.BlockSpec((128, aout * 2), lambda i: (i, 0)),
            pl.BlockSpec((128, aout), lambda i: (i, 0)),
        ],
        out_shape=[
            jax.ShapeDtypeStruct((_QP, aout * 2), f32),
            jax.ShapeDtypeStruct((_QP, aout), f32),
        ],
    )(qpad, W_off.T, b_off.reshape(1, -1), W_attn.T, b_attn.reshape(1, -1))

    off6 = offs.reshape(_QP, _H, _V, _L, _R, _Z, 2)
    off_t = off6.transpose(2, 3, 0, 1, 4, 5, 6).reshape(_VL, _QP, _HRZ, 2)
    offx = off_t[..., 0]
    offy = off_t[..., 1]
    attn6 = attn.reshape(_QP, _H, _V, _L, _R, _Z)
    attn_t = attn6.transpose(2, 3, 0, 1, 4, 5).reshape(_VL, _QP, _HRZ)
    rp = ref_points.reshape(_Q, _V, _L, _Z, 2)
    rp = jnp.pad(rp, ((0, _QP - _Q), (0, 0), (0, 0), (0, 0), (0, 0)))
    rp_t = rp.transpose(1, 2, 0, 3, 4).reshape(_VL, _QP, _Z, 2)
    refx = rp_t[..., 0]
    refy = rp_t[..., 1]

    vlqb = pl.BlockSpec((1, 128, _HRZ), lambda vl, qb: (vl, qb, 0))
    addr_out = pl.pallas_call(
        _addr_kernel,
        grid=(_VL, _QB),
        in_specs=[
            vlqb, vlqb, vlqb,
            pl.BlockSpec((1, 128, _Z), lambda vl, qb: (vl, qb, 0)),
            pl.BlockSpec((1, 128, _Z), lambda vl, qb: (vl, qb, 0)),
        ],
        out_specs=[vlqb] * 5,
        out_shape=[jax.ShapeDtypeStruct((_VL, _QP, _HRZ), jnp.int32)]
        + [jax.ShapeDtypeStruct((_VL, _QP, _HRZ), f32)] * 4,
    )(offx, offy, attn_t, refx, refy)
    lin, w00, w01, w10, w11 = addr_out

    idxcat = lin.reshape(_NROWS)
    wtcat = jnp.stack([w00, w01, w10, w11], axis=-1)
    wtview = wtcat.reshape(_VL, _QP, _H, _RZ, 4)

    rows = _sc_gather_rows(table, idxcat)
    rowsview = rows.reshape(_VL, _QP, _H, _RZ, 4 * _HD)

    acc = pl.pallas_call(
        _reduce_kernel,
        grid=(_QP // 8,),
        in_specs=[
            pl.BlockSpec((_VL, 8, _H, _RZ, 4 * _HD),
                         lambda i: (0, i, 0, 0, 0)),
            pl.BlockSpec((_VL, 8, _H, _RZ, 4), lambda i: (0, i, 0, 0, 0)),
        ],
        out_specs=pl.BlockSpec((8, _D), lambda i: (i, 0)),
        out_shape=jax.ShapeDtypeStruct((_QP, _D), f32),
    )(rowsview, wtview)

    out = pl.pallas_call(
        _oproj_kernel,
        grid=(_QB,),
        in_specs=[
            pl.BlockSpec((128, _D), lambda i: (i, 0)),
            pl.BlockSpec((_D, _D), lambda i: (0, 0)),
        ],
        out_specs=pl.BlockSpec((128, _D), lambda i: (i, 0)),
        out_shape=jax.ShapeDtypeStruct((_QP, _D), f32),
    )(acc, W_out.T)

    return out[:_Q].reshape(_N, _Q, _D)
